# Initial kernel scaffold; baseline (speedup 1.0000x reference)
import jax
import jax.numpy as jnp
from jax.experimental import pallas as pl


def kernel(tokens12, map12to5, map12to6, map12to7, map12to8, map12to9, map12to10, map12to11):
    raise NotImplementedError("write your pallas kernel here")


# speedup vs baseline: 386.2496x; 386.2496x over previous
import jax
import jax.numpy as jnp
from jax import lax
from jax.experimental import pallas as pl
from jax.experimental.pallas import tpu as pltpu
from jax.experimental.pallas import tpu_sc as plsc

NC, NS, L = 2, 16, 16
NW = NC * NS
BATCH, SEQ = 4096, 200
N_TOK = BATCH * SEQ
TOK_PER_W = N_TOK // NW
SUB = 3200
NBLK = TOK_PER_W // SUB
GROUPS = SUB // L
NMAP = 7
TBL = 4096


def _sc_body(tok_hbm, m5, m6, m7, m8, m9, m10, m11, out_hbm,
             tables_v, tok_v, out_v):
    wid = lax.axis_index("s") * NC + lax.axis_index("c")
    base = wid * TOK_PER_W

    for k, m in enumerate((m5, m6, m7, m8, m9, m10, m11)):
        pltpu.sync_copy(m, tables_v.at[pl.ds(k * TBL, TBL)])

    for blk in range(NBLK):
        off = base + blk * SUB
        pltpu.sync_copy(tok_hbm.at[pl.ds(off, SUB)], tok_v)

        def body(g, carry):
            toks = tok_v[pl.ds(g * L, L)]
            for k in range(NMAP):
                vals = plsc.load_gather(tables_v, [toks + (k * TBL)])
                out_v[pl.ds(k * SUB + g * L, L)] = vals
            return carry

        lax.fori_loop(0, GROUPS, body, 0)

        for k in range(NMAP):
            pltpu.sync_copy(out_v.at[pl.ds(k * SUB, SUB)],
                            out_hbm.at[pl.ds(k * N_TOK + off, SUB)])


def kernel(tokens12, map12to5, map12to6, map12to7, map12to8, map12to9,
           map12to10, map12to11):
    tok = tokens12.reshape(-1)
    mesh = plsc.VectorSubcoreMesh(core_axis_name="c", subcore_axis_name="s",
                                  num_cores=NC, num_subcores=NS)
    out = pl.kernel(
        _sc_body,
        out_type=jax.ShapeDtypeStruct((NMAP * N_TOK,), jnp.float32),
        mesh=mesh,
        scratch_types=[
            pltpu.VMEM((NMAP * TBL,), jnp.float32),
            pltpu.VMEM((SUB,), jnp.int32),
            pltpu.VMEM((NMAP * SUB,), jnp.float32),
        ],
        compiler_params=pltpu.CompilerParams(needs_layout_passes=False),
    )(tok, map12to5, map12to6, map12to7, map12to8, map12to9, map12to10,
      map12to11)
    return out.reshape(NMAP, BATCH, SEQ)


# speedup vs baseline: 418.5194x; 1.0835x over previous
import jax
import jax.numpy as jnp
from jax import lax
from jax.experimental import pallas as pl
from jax.experimental.pallas import tpu as pltpu
from jax.experimental.pallas import tpu_sc as plsc

NC, NS, L = 2, 16, 16
NW = NC * NS
BATCH, SEQ = 4096, 200
N_TOK = BATCH * SEQ
TOK_PER_W = N_TOK // NW
SUB = 3200
NBLK = TOK_PER_W // SUB
GROUPS = SUB // L
NMAP = 7
TBL = 4096


def _sc_body(tok_hbm, m5, m6, m7, m8, m9, m10, m11, out_hbm,
             tables_v, tok0, tok1, out0, out1,
             tok_sem0, tok_sem1, out_sem0, out_sem1):
    wid = lax.axis_index("s") * NC + lax.axis_index("c")
    base = wid * TOK_PER_W

    for k, m in enumerate((m5, m6, m7, m8, m9, m10, m11)):
        pltpu.sync_copy(m, tables_v.at[pl.ds(k * TBL, TBL)])

    tok_bufs = (tok0, tok1)
    out_bufs = (out0, out1)
    tok_sems = (tok_sem0, tok_sem1)
    out_sems = (out_sem0, out_sem1)

    tok_copies = [None] * NBLK
    out_copies = [None] * NBLK

    tok_copies[0] = pltpu.async_copy(
        tok_hbm.at[pl.ds(base, SUB)], tok_bufs[0], tok_sems[0])

    for blk in range(NBLK):
        p = blk % 2
        tok_v = tok_bufs[p]
        out_v = out_bufs[p]
        off = base + blk * SUB

        tok_copies[blk].wait()
        if blk + 1 < NBLK:
            nxt = (blk + 1) % 2
            tok_copies[blk + 1] = pltpu.async_copy(
                tok_hbm.at[pl.ds(base + (blk + 1) * SUB, SUB)],
                tok_bufs[nxt], tok_sems[nxt])
        if blk >= 2:
            for c in out_copies[blk - 2]:
                c.wait()

        def body(g, carry):
            toks = tok_v[pl.ds(g * L, L)]
            for k in range(NMAP):
                vals = plsc.load_gather(tables_v, [toks + (k * TBL)])
                out_v[pl.ds(k * SUB + g * L, L)] = vals
            return carry

        lax.fori_loop(0, GROUPS, body, 0)

        out_copies[blk] = [
            pltpu.async_copy(out_v.at[pl.ds(k * SUB, SUB)],
                             out_hbm.at[pl.ds(k * N_TOK + off, SUB)],
                             out_sems[p])
            for k in range(NMAP)
        ]

    for blk in (NBLK - 2, NBLK - 1):
        for c in out_copies[blk]:
            c.wait()


def kernel(tokens12, map12to5, map12to6, map12to7, map12to8, map12to9,
           map12to10, map12to11):
    tok = tokens12.reshape(-1)
    mesh = plsc.VectorSubcoreMesh(core_axis_name="c", subcore_axis_name="s",
                                  num_cores=NC, num_subcores=NS)
    out = pl.kernel(
        _sc_body,
        out_type=jax.ShapeDtypeStruct((NMAP * N_TOK,), jnp.float32),
        mesh=mesh,
        scratch_types=[
            pltpu.VMEM((NMAP * TBL,), jnp.float32),
            pltpu.VMEM((SUB,), jnp.int32),
            pltpu.VMEM((SUB,), jnp.int32),
            pltpu.VMEM((NMAP * SUB,), jnp.float32),
            pltpu.VMEM((NMAP * SUB,), jnp.float32),
            pltpu.SemaphoreType.DMA,
            pltpu.SemaphoreType.DMA,
            pltpu.SemaphoreType.DMA,
            pltpu.SemaphoreType.DMA,
        ],
        compiler_params=pltpu.CompilerParams(needs_layout_passes=False),
    )(tok, map12to5, map12to6, map12to7, map12to8, map12to9, map12to10,
      map12to11)
    return out.reshape(NMAP, BATCH, SEQ)


# speedup vs baseline: 581.7651x; 1.3901x over previous
import jax
import jax.numpy as jnp
from jax import lax
from jax.experimental import pallas as pl
from jax.experimental.pallas import tpu as pltpu
from jax.experimental.pallas import tpu_sc as plsc

NC, NS, L = 2, 16, 16
NW = NC * NS
BATCH, SEQ = 4096, 200
N_TOK = BATCH * SEQ
TOK_PER_W = N_TOK // NW
SUB = 3200
NBLK = TOK_PER_W // SUB
GROUPS = SUB // L
NMAP = 7
TBL = 4096


def _sc_body(tok_hbm, m5, m6, m7, m8, m9, m10, m11, out_hbm,
             tables_v, tok0, tok1, out0, out1,
             tok_sem0, tok_sem1, out_sem0, out_sem1):
    wid = lax.axis_index("s") * NC + lax.axis_index("c")
    base = wid * TOK_PER_W

    for k, m in enumerate((m5, m6, m7, m8, m9, m10, m11)):
        pltpu.sync_copy(m, tables_v.at[pl.ds(k * TBL, TBL)])

    tok_bufs = (tok0, tok1)
    out_bufs = (out0, out1)
    tok_sems = (tok_sem0, tok_sem1)
    out_sems = (out_sem0, out_sem1)

    tok_copies = [None] * NBLK
    out_copies = [None] * NBLK

    tok_copies[0] = pltpu.async_copy(
        tok_hbm.at[pl.ds(base, SUB)], tok_bufs[0], tok_sems[0])

    for blk in range(NBLK):
        p = blk % 2
        tok_v = tok_bufs[p]
        out_v = out_bufs[p]
        off = base + blk * SUB

        tok_copies[blk].wait()
        if blk + 1 < NBLK:
            nxt = (blk + 1) % 2
            tok_copies[blk + 1] = pltpu.async_copy(
                tok_hbm.at[pl.ds(base + (blk + 1) * SUB, SUB)],
                tok_bufs[nxt], tok_sems[nxt])
        if blk >= 2:
            for c in out_copies[blk - 2]:
                c.wait()

        @plsc.parallel_loop(0, GROUPS, 1, unroll=8)
        def _gather(g):
            toks = tok_v[pl.ds(g * L, L)]
            for k in range(NMAP):
                vals = plsc.load_gather(tables_v, [toks + (k * TBL)])
                out_v[pl.ds(k * SUB + g * L, L)] = vals

        out_copies[blk] = [
            pltpu.async_copy(out_v.at[pl.ds(k * SUB, SUB)],
                             out_hbm.at[pl.ds(k * N_TOK + off, SUB)],
                             out_sems[p])
            for k in range(NMAP)
        ]

    for blk in (NBLK - 2, NBLK - 1):
        for c in out_copies[blk]:
            c.wait()


def kernel(tokens12, map12to5, map12to6, map12to7, map12to8, map12to9,
           map12to10, map12to11):
    tok = tokens12.reshape(-1)
    mesh = plsc.VectorSubcoreMesh(core_axis_name="c", subcore_axis_name="s",
                                  num_cores=NC, num_subcores=NS)
    out = pl.kernel(
        _sc_body,
        out_type=jax.ShapeDtypeStruct((NMAP * N_TOK,), jnp.float32),
        mesh=mesh,
        scratch_types=[
            pltpu.VMEM((NMAP * TBL,), jnp.float32),
            pltpu.VMEM((SUB,), jnp.int32),
            pltpu.VMEM((SUB,), jnp.int32),
            pltpu.VMEM((NMAP * SUB,), jnp.float32),
            pltpu.VMEM((NMAP * SUB,), jnp.float32),
            pltpu.SemaphoreType.DMA,
            pltpu.SemaphoreType.DMA,
            pltpu.SemaphoreType.DMA,
            pltpu.SemaphoreType.DMA,
        ],
        compiler_params=pltpu.CompilerParams(needs_layout_passes=False),
    )(tok, map12to5, map12to6, map12to7, map12to8, map12to9, map12to10,
      map12to11)
    return out.reshape(NMAP, BATCH, SEQ)


# speedup vs baseline: 858.9724x; 1.4765x over previous
import jax
import jax.numpy as jnp
from jax import lax
from jax.experimental import pallas as pl
from jax.experimental.pallas import tpu as pltpu
from jax.experimental.pallas import tpu_sc as plsc

NC, NS, L = 2, 16, 16
NW = NC * NS
BATCH, SEQ = 4096, 200
ROWS_PER_W = BATCH // NW
RBLK = 16
NBLK = ROWS_PER_W // RBLK
NMAP = 7
TBL = 4096
COL_STARTS = tuple(range(0, SEQ - L + 1, L)) + (SEQ - L,)


def _sc_body(tok_hbm, m5, m6, m7, m8, m9, m10, m11, out_hbm,
             tables_v, tok0, tok1, out0, out1,
             tok_sem0, tok_sem1, out_sem0, out_sem1):
    wid = lax.axis_index("s") * NC + lax.axis_index("c")
    row0 = wid * ROWS_PER_W

    for k, m in enumerate((m5, m6, m7, m8, m9, m10, m11)):
        pltpu.sync_copy(m, tables_v.at[pl.ds(k * TBL, TBL)])

    tok_bufs = (tok0, tok1)
    out_bufs = (out0, out1)
    tok_sems = (tok_sem0, tok_sem1)
    out_sems = (out_sem0, out_sem1)

    tok_copies = [None] * NBLK
    out_copies = [None] * NBLK

    tok_copies[0] = pltpu.async_copy(
        tok_hbm.at[pl.ds(row0, RBLK), :], tok_bufs[0], tok_sems[0])

    for blk in range(NBLK):
        p = blk % 2
        tok_v = tok_bufs[p]
        out_v = out_bufs[p]
        r_off = row0 + blk * RBLK

        tok_copies[blk].wait()
        if blk + 1 < NBLK:
            nxt = (blk + 1) % 2
            tok_copies[blk + 1] = pltpu.async_copy(
                tok_hbm.at[pl.ds(row0 + (blk + 1) * RBLK, RBLK), :],
                tok_bufs[nxt], tok_sems[nxt])
        if blk >= 2:
            for c in out_copies[blk - 2]:
                c.wait()

        @plsc.parallel_loop(0, RBLK, 1)
        def _gather(r):
            @plsc.parallel_loop(0, SEQ - L, L, unroll=2)
            def _cols(c):
                toks = tok_v[r, pl.ds(c, L)]
                for k in range(NMAP):
                    vals = plsc.load_gather(tables_v, [toks + (k * TBL)])
                    out_v[k, r, pl.ds(c, L)] = vals

            toks = tok_v[r, pl.ds(SEQ - L, L)]
            for k in range(NMAP):
                vals = plsc.load_gather(tables_v, [toks + (k * TBL)])
                out_v[k, r, pl.ds(SEQ - L, L)] = vals

        out_copies[blk] = [
            pltpu.async_copy(out_v.at[k],
                             out_hbm.at[k, pl.ds(r_off, RBLK), :],
                             out_sems[p])
            for k in range(NMAP)
        ]

    for blk in (NBLK - 2, NBLK - 1):
        for c in out_copies[blk]:
            c.wait()


def kernel(tokens12, map12to5, map12to6, map12to7, map12to8, map12to9,
           map12to10, map12to11):
    mesh = plsc.VectorSubcoreMesh(core_axis_name="c", subcore_axis_name="s",
                                  num_cores=NC, num_subcores=NS)
    out = pl.kernel(
        _sc_body,
        out_type=jax.ShapeDtypeStruct((NMAP, BATCH, SEQ), jnp.float32),
        mesh=mesh,
        scratch_types=[
            pltpu.VMEM((NMAP * TBL,), jnp.float32),
            pltpu.VMEM((RBLK, SEQ), jnp.int32),
            pltpu.VMEM((RBLK, SEQ), jnp.int32),
            pltpu.VMEM((NMAP, RBLK, SEQ), jnp.float32),
            pltpu.VMEM((NMAP, RBLK, SEQ), jnp.float32),
            pltpu.SemaphoreType.DMA,
            pltpu.SemaphoreType.DMA,
            pltpu.SemaphoreType.DMA,
            pltpu.SemaphoreType.DMA,
        ],
        compiler_params=pltpu.CompilerParams(needs_layout_passes=False),
    )(tokens12, map12to5, map12to6, map12to7, map12to8, map12to9, map12to10,
      map12to11)
    return out


# speedup vs baseline: 864.4623x; 1.0064x over previous
import jax
import jax.numpy as jnp
from jax import lax
from jax.experimental import pallas as pl
from jax.experimental.pallas import tpu as pltpu
from jax.experimental.pallas import tpu_sc as plsc

NC, NS, L = 2, 16, 16
NW = NC * NS
BATCH, SEQ = 4096, 200
ROWS_PER_W = BATCH // NW
RBLK = 16
NBLK = ROWS_PER_W // RBLK
NMAP = 7
TBL = 4096
COL_STARTS = tuple(range(0, SEQ - L + 1, L)) + (SEQ - L,)


def _sc_body(tok_hbm, m5, m6, m7, m8, m9, m10, m11, out_hbm,
             tables_v, tok0, tok1, out0, out1,
             tok_sem0, tok_sem1, out_sem0, out_sem1):
    wid = lax.axis_index("s") * NC + lax.axis_index("c")
    row0 = wid * ROWS_PER_W

    for k, m in enumerate((m5, m6, m7, m8, m9, m10, m11)):
        pltpu.sync_copy(m, tables_v.at[pl.ds(k * TBL, TBL)])

    tok_bufs = (tok0, tok1)
    out_bufs = (out0, out1)
    tok_sems = (tok_sem0, tok_sem1)
    out_sems = (out_sem0, out_sem1)

    tok_copies = [None] * NBLK
    out_copies = [None] * NBLK

    tok_copies[0] = pltpu.async_copy(
        tok_hbm.at[pl.ds(row0, RBLK), :], tok_bufs[0], tok_sems[0])

    for blk in range(NBLK):
        p = blk % 2
        tok_v = tok_bufs[p]
        out_v = out_bufs[p]
        r_off = row0 + blk * RBLK

        tok_copies[blk].wait()
        if blk + 1 < NBLK:
            nxt = (blk + 1) % 2
            tok_copies[blk + 1] = pltpu.async_copy(
                tok_hbm.at[pl.ds(row0 + (blk + 1) * RBLK, RBLK), :],
                tok_bufs[nxt], tok_sems[nxt])
        if blk >= 2:
            for c in out_copies[blk - 2]:
                c.wait()

        @plsc.parallel_loop(0, RBLK, 1)
        def _gather(r):
            @plsc.parallel_loop(0, SEQ - L, L, unroll=4)
            def _cols(c):
                toks = tok_v[r, pl.ds(c, L)]
                for k in range(NMAP):
                    vals = plsc.load_gather(tables_v, [toks + (k * TBL)])
                    out_v[k, r, pl.ds(c, L)] = vals

            toks = tok_v[r, pl.ds(SEQ - L, L)]
            for k in range(NMAP):
                vals = plsc.load_gather(tables_v, [toks + (k * TBL)])
                out_v[k, r, pl.ds(SEQ - L, L)] = vals

        out_copies[blk] = [
            pltpu.async_copy(out_v.at[k],
                             out_hbm.at[k, pl.ds(r_off, RBLK), :],
                             out_sems[p])
            for k in range(NMAP)
        ]

    for blk in (NBLK - 2, NBLK - 1):
        for c in out_copies[blk]:
            c.wait()


def kernel(tokens12, map12to5, map12to6, map12to7, map12to8, map12to9,
           map12to10, map12to11):
    mesh = plsc.VectorSubcoreMesh(core_axis_name="c", subcore_axis_name="s",
                                  num_cores=NC, num_subcores=NS)
    out = pl.kernel(
        _sc_body,
        out_type=jax.ShapeDtypeStruct((NMAP, BATCH, SEQ), jnp.float32),
        mesh=mesh,
        scratch_types=[
            pltpu.VMEM((NMAP * TBL,), jnp.float32),
            pltpu.VMEM((RBLK, SEQ), jnp.int32),
            pltpu.VMEM((RBLK, SEQ), jnp.int32),
            pltpu.VMEM((NMAP, RBLK, SEQ), jnp.float32),
            pltpu.VMEM((NMAP, RBLK, SEQ), jnp.float32),
            pltpu.SemaphoreType.DMA,
            pltpu.SemaphoreType.DMA,
            pltpu.SemaphoreType.DMA,
            pltpu.SemaphoreType.DMA,
        ],
        compiler_params=pltpu.CompilerParams(needs_layout_passes=False),
    )(tokens12, map12to5, map12to6, map12to7, map12to8, map12to9, map12to10,
      map12to11)
    return out


# speedup vs baseline: 870.3412x; 1.0068x over previous
import jax
import jax.numpy as jnp
from jax import lax
from jax.experimental import pallas as pl
from jax.experimental.pallas import tpu as pltpu
from jax.experimental.pallas import tpu_sc as plsc

NC, NS, L = 2, 16, 16
NW = NC * NS
BATCH, SEQ = 4096, 200
ROWS_PER_W = BATCH // NW
RBLK = 16
NBLK = ROWS_PER_W // RBLK
NMAP = 7
TBL = 4096


def _sc_body(tok_hbm, m5, m6, m7, m8, m9, m10, m11, out_hbm,
             tables_v, tok0, tok1, out0, out1,
             tok_sem0, tok_sem1, out_sem0, out_sem1):
    wid = lax.axis_index("s") * NC + lax.axis_index("c")
    row0 = wid * ROWS_PER_W

    for k, m in enumerate((m5, m6, m7, m8, m9, m10, m11)):
        pltpu.sync_copy(m, tables_v.at[pl.ds(k * TBL, TBL)])

    tok_bufs = (tok0, tok1)
    out_bufs = (out0, out1)
    tok_sems = (tok_sem0, tok_sem1)
    out_sems = (out_sem0, out_sem1)

    tok_copies = [None] * NBLK
    out_copies = [None] * NBLK

    tok_copies[0] = pltpu.async_copy(
        tok_hbm.at[pl.ds(row0, RBLK), :], tok_bufs[0], tok_sems[0])

    for blk in range(NBLK):
        p = blk % 2
        tok_v = tok_bufs[p]
        out_v = out_bufs[p]
        r_off = row0 + blk * RBLK

        tok_copies[blk].wait()
        if blk + 1 < NBLK:
            nxt = (blk + 1) % 2
            tok_copies[blk + 1] = pltpu.async_copy(
                tok_hbm.at[pl.ds(row0 + (blk + 1) * RBLK, RBLK), :],
                tok_bufs[nxt], tok_sems[nxt])
        if blk >= 2:
            out_copies[blk - 2].wait()

        @plsc.parallel_loop(0, RBLK, 1)
        def _gather(r):
            @plsc.parallel_loop(0, SEQ - L, L, unroll=4)
            def _cols(c):
                toks = tok_v[r, pl.ds(c, L)]
                for k in range(NMAP):
                    vals = plsc.load_gather(tables_v, [toks + (k * TBL)])
                    out_v[k, r, pl.ds(c, L)] = vals

            toks = tok_v[r, pl.ds(SEQ - L, L)]
            for k in range(NMAP):
                vals = plsc.load_gather(tables_v, [toks + (k * TBL)])
                out_v[k, r, pl.ds(SEQ - L, L)] = vals

        out_copies[blk] = pltpu.async_copy(
            out_v, out_hbm.at[:, pl.ds(r_off, RBLK), :], out_sems[p])

    for blk in (NBLK - 2, NBLK - 1):
        out_copies[blk].wait()


def kernel(tokens12, map12to5, map12to6, map12to7, map12to8, map12to9,
           map12to10, map12to11):
    mesh = plsc.VectorSubcoreMesh(core_axis_name="c", subcore_axis_name="s",
                                  num_cores=NC, num_subcores=NS)
    out = pl.kernel(
        _sc_body,
        out_type=jax.ShapeDtypeStruct((NMAP, BATCH, SEQ), jnp.float32),
        mesh=mesh,
        scratch_types=[
            pltpu.VMEM((NMAP * TBL,), jnp.float32),
            pltpu.VMEM((RBLK, SEQ), jnp.int32),
            pltpu.VMEM((RBLK, SEQ), jnp.int32),
            pltpu.VMEM((NMAP, RBLK, SEQ), jnp.float32),
            pltpu.VMEM((NMAP, RBLK, SEQ), jnp.float32),
            pltpu.SemaphoreType.DMA,
            pltpu.SemaphoreType.DMA,
            pltpu.SemaphoreType.DMA,
            pltpu.SemaphoreType.DMA,
        ],
        compiler_params=pltpu.CompilerParams(needs_layout_passes=False),
    )(tokens12, map12to5, map12to6, map12to7, map12to8, map12to9, map12to10,
      map12to11)
    return out


# speedup vs baseline: 907.4068x; 1.0426x over previous
import jax
import jax.numpy as jnp
from jax import lax
from jax.experimental import pallas as pl
from jax.experimental.pallas import tpu as pltpu
from jax.experimental.pallas import tpu_sc as plsc

NC, NS, L = 2, 16, 16
NW = NC * NS
BATCH, SEQ = 4096, 200
ROWS_PER_W = BATCH // NW
RBLK = 16
NBLK = ROWS_PER_W // RBLK
NMAP = 7
TBL = 4096


def _sc_body(tok_hbm, m5, m6, m7, m8, m9, m10, m11, out_hbm,
             tables_v, tok0, tok1, out0, out1,
             tok_sem0, tok_sem1, out_sem0, out_sem1):
    wid = lax.axis_index("s") * NC + lax.axis_index("c")
    row0 = wid * ROWS_PER_W

    tok_bufs = (tok0, tok1)
    out_bufs = (out0, out1)
    tok_sems = (tok_sem0, tok_sem1)
    out_sems = (out_sem0, out_sem1)

    tok_copies = [None] * NBLK
    out_copies = [None] * NBLK

    tok_copies[0] = pltpu.async_copy(
        tok_hbm.at[pl.ds(row0, RBLK), :], tok_bufs[0], tok_sems[0])

    table_copies = [
        pltpu.async_copy(m, tables_v.at[pl.ds(k * TBL, TBL)], out_sems[1])
        for k, m in enumerate((m5, m6, m7, m8, m9, m10, m11))
    ]
    for c in table_copies:
        c.wait()

    for blk in range(NBLK):
        p = blk % 2
        tok_v = tok_bufs[p]
        out_v = out_bufs[p]
        r_off = row0 + blk * RBLK

        tok_copies[blk].wait()
        if blk + 1 < NBLK:
            nxt = (blk + 1) % 2
            tok_copies[blk + 1] = pltpu.async_copy(
                tok_hbm.at[pl.ds(row0 + (blk + 1) * RBLK, RBLK), :],
                tok_bufs[nxt], tok_sems[nxt])
        if blk >= 2:
            out_copies[blk - 2].wait()

        @plsc.parallel_loop(0, RBLK, 1)
        def _gather(r):
            @plsc.parallel_loop(0, SEQ - L, L, unroll=4)
            def _cols(c):
                toks = tok_v[r, pl.ds(c, L)]
                for k in range(NMAP):
                    vals = plsc.load_gather(tables_v, [toks + (k * TBL)])
                    out_v[k, r, pl.ds(c, L)] = vals

            toks = tok_v[r, pl.ds(SEQ - L, L)]
            for k in range(NMAP):
                vals = plsc.load_gather(tables_v, [toks + (k * TBL)])
                out_v[k, r, pl.ds(SEQ - L, L)] = vals

        out_copies[blk] = pltpu.async_copy(
            out_v, out_hbm.at[:, pl.ds(r_off, RBLK), :], out_sems[p])

    for blk in (NBLK - 2, NBLK - 1):
        out_copies[blk].wait()


def kernel(tokens12, map12to5, map12to6, map12to7, map12to8, map12to9,
           map12to10, map12to11):
    mesh = plsc.VectorSubcoreMesh(core_axis_name="c", subcore_axis_name="s",
                                  num_cores=NC, num_subcores=NS)
    out = pl.kernel(
        _sc_body,
        out_type=jax.ShapeDtypeStruct((NMAP, BATCH, SEQ), jnp.float32),
        mesh=mesh,
        scratch_types=[
            pltpu.VMEM((NMAP * TBL,), jnp.float32),
            pltpu.VMEM((RBLK, SEQ), jnp.int32),
            pltpu.VMEM((RBLK, SEQ), jnp.int32),
            pltpu.VMEM((NMAP, RBLK, SEQ), jnp.float32),
            pltpu.VMEM((NMAP, RBLK, SEQ), jnp.float32),
            pltpu.SemaphoreType.DMA,
            pltpu.SemaphoreType.DMA,
            pltpu.SemaphoreType.DMA,
            pltpu.SemaphoreType.DMA,
        ],
        compiler_params=pltpu.CompilerParams(needs_layout_passes=False),
    )(tokens12, map12to5, map12to6, map12to7, map12to8, map12to9, map12to10,
      map12to11)
    return out
